# baseline (speedup 1.0000x reference)
import functools

import jax
import jax.numpy as jnp
from jax.experimental import pallas as pl
from jax.experimental.pallas import tpu as pltpu
from jax.experimental.pallas import tpu_sc as plsc


_GATHER_WINDOW = 128


@functools.partial(jax.jit, static_argnames=("n_rows",))
def _sc_gather(table, idx_2d, n_rows):
    dim = table.shape[1]
    mesh = plsc.VectorSubcoreMesh(core_axis_name="core", subcore_axis_name="subcore")

    @pl.kernel(
        out_type=jax.ShapeDtypeStruct((n_rows, dim), table.dtype),
        mesh=mesh,
    )
    def gather_kernel(tbl_hbm, i_hbm, o_hbm):
        def body(i_vmem, o_vmem):
            pltpu.sync_copy(tbl_hbm.at[i_vmem.at[0]], o_vmem)

        pltpu.emit_pipeline(
            body,
            grid=(n_rows // _GATHER_WINDOW,),
            in_specs=[pl.BlockSpec((1, _GATHER_WINDOW), index_map=lambda i: (0, i))],
            out_specs=[pl.BlockSpec((_GATHER_WINDOW, dim), index_map=lambda i: (i, 0))],
            core_axis_name=("core", "subcore"),
            dimension_semantics=(pltpu.PARALLEL,),
        )(i_hbm, o_hbm)

    return gather_kernel(table, idx_2d)


def _tc_body(nchunk, cw, vocab, dim, embs_ref, w_ref, b_ref, out_ref):
    pooled = jnp.mean(embs_ref[...], axis=0)[:, :dim]
    pooled_bf = pooled.astype(jnp.bfloat16)

    bblk = pooled.shape[0]
    m = jnp.full((bblk, 1), -jnp.inf, dtype=jnp.float32)
    l = jnp.zeros((bblk, 1), dtype=jnp.float32)
    for j in range(nchunk):
        logits = (
            jnp.dot(pooled_bf, w_ref[j], preferred_element_type=jnp.float32)
            + b_ref[j]
        )
        width = min(cw, vocab - j * cw)
        out_ref[:, j * cw : j * cw + width] = logits[:, :width]
        mj = jnp.max(logits, axis=1, keepdims=True)
        m_new = jnp.maximum(m, mj)
        l = l * jnp.exp(m - m_new) + jnp.sum(
            jnp.exp(logits - m_new), axis=1, keepdims=True
        )
        m = m_new
    lse = m + jnp.log(l)
    out_ref[...] = out_ref[...] - lse


def kernel(inputs, table, W, b):
    batch, ctx = inputs.shape
    dim, vocab = W.shape

    gdim = 128
    table_p = jnp.pad(table, ((0, 0), (0, gdim - dim)))
    idx = jnp.transpose(inputs).reshape(1, batch * ctx).astype(jnp.int32)
    embs = _sc_gather(table_p, idx, n_rows=batch * ctx)
    embs = embs.reshape(ctx, batch, gdim)

    cw = 12544
    nchunk = -(-vocab // cw)
    vpad = nchunk * cw
    w3 = (
        jnp.pad(W, ((0, 0), (0, vpad - vocab)))
        .astype(jnp.bfloat16)
        .reshape(dim, nchunk, cw)
        .transpose(1, 0, 2)
    )
    b3 = jnp.pad(b, (0, vpad - vocab), constant_values=-1e30).reshape(nchunk, 1, cw)

    bblk = 32
    body = functools.partial(_tc_body, nchunk, cw, vocab, dim)
    out = pl.pallas_call(
        body,
        grid=(batch // bblk,),
        in_specs=[
            pl.BlockSpec((ctx, bblk, gdim), lambda i: (0, i, 0)),
            pl.BlockSpec((nchunk, dim, cw), lambda i: (0, 0, 0)),
            pl.BlockSpec((nchunk, 1, cw), lambda i: (0, 0, 0)),
        ],
        out_specs=pl.BlockSpec((bblk, vocab), lambda i: (i, 0)),
        out_shape=jax.ShapeDtypeStruct((batch, vocab), jnp.float32),
    )(embs, w3, b3)
    return out


# speedup vs baseline: 2.4050x; 2.4050x over previous
import functools

import jax
import jax.numpy as jnp
from jax.experimental import pallas as pl
from jax.experimental.pallas import tpu as pltpu
from jax.experimental.pallas import tpu_sc as plsc


_GATHER_WINDOW = 128


@functools.partial(jax.jit, static_argnames=("n_rows",))
def _sc_gather(table, idx_2d, n_rows):
    dim = table.shape[1]
    mesh = plsc.VectorSubcoreMesh(core_axis_name="core", subcore_axis_name="subcore")

    @pl.kernel(
        out_type=jax.ShapeDtypeStruct((n_rows, dim), table.dtype),
        mesh=mesh,
    )
    def gather_kernel(tbl_hbm, i_hbm, o_hbm):
        def body(i_vmem, o_vmem):
            pltpu.sync_copy(tbl_hbm.at[i_vmem.at[0]], o_vmem)

        pltpu.emit_pipeline(
            body,
            grid=(n_rows // _GATHER_WINDOW,),
            in_specs=[pl.BlockSpec((1, _GATHER_WINDOW), index_map=lambda i: (0, i))],
            out_specs=[pl.BlockSpec((_GATHER_WINDOW, dim), index_map=lambda i: (i, 0))],
            core_axis_name=("core", "subcore"),
            dimension_semantics=(pltpu.PARALLEL,),
        )(i_hbm, o_hbm)

    return gather_kernel(table, idx_2d)


def _stats_body(nchunk, dim, embs_ref, w_ref, b_ref, pooled_ref, lse_ref):
    pooled = jnp.mean(embs_ref[...], axis=0)[:, :dim]
    pooled_ref[...] = pooled
    pooled_bf = pooled.astype(jnp.bfloat16)

    bblk = pooled.shape[0]
    l = jnp.zeros((bblk, 1), dtype=jnp.float32)
    for j in range(nchunk):
        logits = (
            jnp.dot(pooled_bf, w_ref[j], preferred_element_type=jnp.float32)
            + b_ref[j]
        )
        l = l + jnp.sum(jnp.exp(logits), axis=1, keepdims=True)
    lse_ref[...] = jnp.log(l)


def _write_body(wt_ref, pooled_ref, b_ref, lse_ref, out_ref):
    out_ref[...] = (
        jnp.dot(wt_ref[...], pooled_ref[...], preferred_element_type=jnp.float32)
        + b_ref[...]
        - lse_ref[...]
    )


def kernel(inputs, table, W, b):
    batch, ctx = inputs.shape
    dim, vocab = W.shape

    gdim = 128
    table_p = jnp.pad(table, ((0, 0), (0, gdim - dim)))
    idx = jnp.transpose(inputs).reshape(1, batch * ctx).astype(jnp.int32)
    embs = _sc_gather(table_p, idx, n_rows=batch * ctx)
    embs = embs.reshape(ctx, batch, gdim)

    cw = 12544
    nchunk = -(-vocab // cw)
    vpad = nchunk * cw
    w3 = (
        jnp.pad(W, ((0, 0), (0, vpad - vocab)))
        .astype(jnp.bfloat16)
        .reshape(dim, nchunk, cw)
        .transpose(1, 0, 2)
    )
    b3 = jnp.pad(b, (0, vpad - vocab), constant_values=-1e30).reshape(nchunk, 1, cw)

    bblk1 = 256
    pooled, lse = pl.pallas_call(
        functools.partial(_stats_body, nchunk, dim),
        grid=(batch // bblk1,),
        in_specs=[
            pl.BlockSpec((ctx, bblk1, gdim), lambda i: (0, i, 0)),
            pl.BlockSpec((nchunk, dim, cw), lambda i: (0, 0, 0)),
            pl.BlockSpec((nchunk, 1, cw), lambda i: (0, 0, 0)),
        ],
        out_specs=[
            pl.BlockSpec((bblk1, dim), lambda i: (i, 0)),
            pl.BlockSpec((bblk1, 1), lambda i: (i, 0)),
        ],
        out_shape=[
            jax.ShapeDtypeStruct((batch, dim), jnp.float32),
            jax.ShapeDtypeStruct((batch, 1), jnp.float32),
        ],
    )(embs, w3, b3)

    wt = jnp.transpose(W).astype(jnp.bfloat16)
    pooled_t = jnp.transpose(pooled).astype(jnp.bfloat16)
    lse_row = lse.reshape(1, batch)
    b2 = b.reshape(vocab, 1)

    vc = 10000
    bblk2 = 512
    out_t = pl.pallas_call(
        _write_body,
        grid=(vocab // vc, batch // bblk2),
        in_specs=[
            pl.BlockSpec((vc, dim), lambda v, i: (v, 0)),
            pl.BlockSpec((dim, bblk2), lambda v, i: (0, i)),
            pl.BlockSpec((vc, 1), lambda v, i: (v, 0)),
            pl.BlockSpec((1, bblk2), lambda v, i: (0, i)),
        ],
        out_specs=pl.BlockSpec((vc, bblk2), lambda v, i: (v, i)),
        out_shape=jax.ShapeDtypeStruct((vocab, batch), jnp.float32),
    )(wt, pooled_t, b2, lse_row)
    return jnp.transpose(out_t)
